# baseline (speedup 1.0000x reference)
import functools

import jax
import jax.numpy as jnp
from jax import lax
from jax.experimental import pallas as pl
from jax.experimental.pallas import tpu as pltpu
from jax.experimental.pallas import tpu_sc as plsc

NC = 2
NS = 16
NW = NC * NS
L = 16


def _make_transpose(V, D):
    CH = 160
    n_ch = V // CH
    slots = (n_ch + NW - 1) // NW
    pad = 9
    mesh = plsc.VectorSubcoreMesh(core_axis_name="c", subcore_axis_name="s")

    @functools.partial(
        pl.kernel,
        mesh=mesh,
        out_type=jax.ShapeDtypeStruct((V, D), jnp.float32),
        scratch_types=[
            pltpu.VMEM((2, D, CH), jnp.float32),
            pltpu.VMEM((2, CH, D + pad), jnp.float32),
            pltpu.SemaphoreType.DMA((2,)),
            pltpu.SemaphoreType.DMA((2,)),
        ],
        compiler_params=pltpu.CompilerParams(
            use_tc_tiling_on_sc=False, needs_layout_passes=False
        ),
    )
    def k(wv_hbm, out_hbm, slab, bt, sem_g, sem_s):
        wid = lax.axis_index("s") * NC + lax.axis_index("c")

        rows = [lax.iota(jnp.int32, L) + (g * L) for g in range(CH // L)]

        def cid_of(t):
            return t * NW + wid

        def fire(t, p):
            pltpu.async_copy(
                wv_hbm.at[:, pl.ds(cid_of(t) * CH, CH)], slab.at[p],
                sem_g.at[p],
            )

        def wait_load(p):
            pltpu.make_async_copy(
                wv_hbm.at[:, pl.ds(0, CH)], slab.at[p], sem_g.at[p]
            ).wait()

        def transpose(p):
            for d in range(D):
                cols = jnp.full((L,), d, jnp.int32)
                for g in range(CH // L):
                    v = slab[p, d, pl.ds(g * L, L)]
                    plsc.store_scatter(bt.at[p], [rows[g], cols], v)

        def store(t, p):
            pltpu.async_copy(
                bt.at[p, :, pl.ds(0, D)],
                out_hbm.at[pl.ds(cid_of(t) * CH, CH)],
                sem_s.at[p],
            )

        def wait_store(p):
            pltpu.make_async_copy(
                bt.at[p, :, pl.ds(0, D)],
                out_hbm.at[pl.ds(0, CH)],
                sem_s.at[p],
            ).wait()

        def valid(t):
            return cid_of(t) < n_ch

        for p in range(2):
            @pl.when(valid(p))
            def _():
                fire(p, p)

        def body(i, carry):
            for sub in range(2):
                t = i * 2 + sub

                @pl.when(valid(t))
                def _():
                    wait_load(sub)

                    @pl.when(i > 0)
                    def _():
                        wait_store(sub)

                    transpose(sub)

                    @pl.when(valid(t + 2))
                    def _():
                        fire(t + 2, sub)

                    store(t, sub)

            return carry

        lax.fori_loop(0, slots // 2, body, 0, unroll=False)
        for p in range(2):
            @pl.when(valid(slots - 2 + p))
            def _():
                wait_store(p)

    return k


def _make_gather(B, H, V, D):
    bw = B // NW
    ng = bw // L
    mesh = plsc.VectorSubcoreMesh(core_axis_name="c", subcore_axis_name="s")

    @functools.partial(
        pl.kernel,
        mesh=mesh,
        out_type=jax.ShapeDtypeStruct((H, D, B), jnp.float32),
        scratch_types=[
            pltpu.VMEM((H, bw), jnp.int32),
            pltpu.VMEM((4, bw, D), jnp.float32),
            pltpu.VMEM((2, D, bw + 9), jnp.float32),
            pltpu.SemaphoreType.DMA((4,)),
            pltpu.SemaphoreType.DMA((2,)),
        ],
        compiler_params=pltpu.CompilerParams(
            use_tc_tiling_on_sc=False, needs_layout_passes=False
        ),
    )
    def k(idx_hbm, table_hbm, out_hbm, idx_v, buf, bt, sem_g, sem_s):
        wid = lax.axis_index("s") * NC + lax.axis_index("c")
        col0 = wid * bw
        pltpu.sync_copy(idx_hbm.at[:, pl.ds(col0, bw)], idx_v)

        rows_lo = lax.iota(jnp.int32, L)
        rows_hi = rows_lo + L

        def fire(h, p):
            pltpu.async_copy(
                table_hbm.at[idx_v.at[h]], buf.at[p], sem_g.at[p]
            )

        def wait_gather(p):
            pltpu.make_async_copy(
                table_hbm.at[pl.ds(0, bw)], buf.at[p], sem_g.at[p]
            ).wait()

        def transpose(p, q):
            def body(c, carry):
                cols = jnp.full((L,), c, jnp.int32)
                v_lo = buf[p, c, pl.ds(0, L)]
                v_hi = buf[p, c, pl.ds(L, L)]
                plsc.store_scatter(bt.at[q], [rows_lo, cols], v_lo)
                plsc.store_scatter(bt.at[q], [rows_hi, cols], v_hi)
                return carry

            lax.fori_loop(0, bw, body, 0, unroll=8)

        def store(h, q):
            pltpu.async_copy(
                bt.at[q, :, pl.ds(0, bw)],
                out_hbm.at[h, :, pl.ds(col0, bw)],
                sem_s.at[q],
            )

        def wait_store(q):
            pltpu.make_async_copy(
                bt.at[q, :, pl.ds(0, bw)],
                out_hbm.at[0, :, pl.ds(col0, bw)],
                sem_s.at[q],
            ).wait()

        for p in range(4):
            fire(p, p)

        def body(i, carry):
            h0 = i * 4
            for p in range(4):
                q = p % 2
                wait_gather(p)

                if p >= 2:
                    wait_store(q)
                else:

                    @pl.when(i > 0)
                    def _():
                        wait_store(q)

                transpose(p, q)

                @pl.when(i < H // 4 - 1)
                def _():
                    fire(h0 + 4 + p, p)

                store(h0 + p, q)
            return carry

        lax.fori_loop(0, H // 4, body, 0, unroll=False)
        wait_store(0)
        wait_store(1)

    return k


def kernel(x, W):
    B, H = x.shape
    V, D = W.shape
    w_lin = _make_transpose(V, D)(W.T)
    out_t = _make_gather(B, H, V, D)(x.T.astype(jnp.int32), w_lin)
    return jnp.transpose(out_t, (2, 0, 1))


# speedup vs baseline: 3.8079x; 3.8079x over previous
import functools

import jax
import jax.numpy as jnp
from jax import lax
from jax.experimental import pallas as pl
from jax.experimental.pallas import tpu as pltpu
from jax.experimental.pallas import tpu_sc as plsc

NC = 2
NS = 16
NW = NC * NS
L = 16


def _make_gather(B, H, V, D):
    bw = B // NW
    ng = bw // L
    mesh = plsc.VectorSubcoreMesh(core_axis_name="c", subcore_axis_name="s")

    @functools.partial(
        pl.kernel,
        mesh=mesh,
        out_type=jax.ShapeDtypeStruct((H, D, B), jnp.float32),
        scratch_types=[
            pltpu.VMEM((H, bw), jnp.int32),
            pltpu.VMEM((4, bw, D), jnp.float32),
            pltpu.VMEM((2, D, bw + 9), jnp.float32),
            pltpu.SemaphoreType.DMA((4,)),
            pltpu.SemaphoreType.DMA((2,)),
        ],
        compiler_params=pltpu.CompilerParams(
            use_tc_tiling_on_sc=False, needs_layout_passes=False
        ),
    )
    def k(idx_hbm, table_hbm, out_hbm, idx_v, buf, bt, sem_g, sem_s):
        wid = lax.axis_index("s") * NC + lax.axis_index("c")
        col0 = wid * bw
        pltpu.sync_copy(idx_hbm.at[:, pl.ds(col0, bw)], idx_v)

        rows_lo = lax.iota(jnp.int32, L)
        rows_hi = rows_lo + L

        def fire(h, p):
            pltpu.async_copy(
                table_hbm.at[idx_v.at[h]], buf.at[p], sem_g.at[p]
            )

        def wait_gather(p):
            pltpu.make_async_copy(
                table_hbm.at[pl.ds(0, bw)], buf.at[p], sem_g.at[p]
            ).wait()

        def transpose(p, q):
            def body(c, carry):
                cols = jnp.full((L,), c, jnp.int32)
                v_lo = buf[p, c, pl.ds(0, L)]
                v_hi = buf[p, c, pl.ds(L, L)]
                plsc.store_scatter(bt.at[q], [rows_lo, cols], v_lo)
                plsc.store_scatter(bt.at[q], [rows_hi, cols], v_hi)
                return carry

            lax.fori_loop(0, bw, body, 0, unroll=8)

        def store(h, q):
            pltpu.async_copy(
                bt.at[q, :, pl.ds(0, bw)],
                out_hbm.at[h, :, pl.ds(col0, bw)],
                sem_s.at[q],
            )

        def wait_store(q):
            pltpu.make_async_copy(
                bt.at[q, :, pl.ds(0, bw)],
                out_hbm.at[0, :, pl.ds(col0, bw)],
                sem_s.at[q],
            ).wait()

        for p in range(4):
            fire(p, p)

        def body(i, carry):
            h0 = i * 4
            for p in range(4):
                q = p % 2
                wait_gather(p)

                if p >= 2:
                    wait_store(q)
                else:

                    @pl.when(i > 0)
                    def _():
                        wait_store(q)

                transpose(p, q)

                @pl.when(i < H // 4 - 1)
                def _():
                    fire(h0 + 4 + p, p)

                store(h0 + p, q)
            return carry

        lax.fori_loop(0, H // 4, body, 0, unroll=False)
        wait_store(0)
        wait_store(1)

    return k


def kernel(x, W):
    B, H = x.shape
    V, D = W.shape
    out_t = _make_gather(B, H, V, D)(x.T.astype(jnp.int32), W)
    return jnp.transpose(out_t, (2, 0, 1))
